# baseline (speedup 1.0000x reference)
import functools

import jax
import jax.numpy as jnp
from jax import lax
from jax.experimental import pallas as pl
from jax.experimental.pallas import tpu as pltpu
from jax.experimental.pallas import tpu_sc as plsc

VOCAB = 1000
EMBED = 1000
BATCH = 1024
SEQ = 20
EMBED_PAD = 1024

CHUNK = 32


@functools.lru_cache(maxsize=None)
def _build_gather(total, embed_pad):
    info = plsc.get_sparse_core_info()
    nc, ns = info.num_cores, info.num_subcores
    nw = nc * ns
    rpw = total // nw
    assert rpw * nw == total
    nchunk = rpw // CHUNK
    assert nchunk * CHUNK == rpw

    mesh = plsc.VectorSubcoreMesh(core_axis_name="c", subcore_axis_name="s")

    @functools.partial(
        pl.kernel,
        mesh=mesh,
        out_type=jax.ShapeDtypeStruct((total, embed_pad), jnp.float32),
        scratch_types=[
            pltpu.VMEM((rpw,), jnp.int32),
            pltpu.VMEM((2, CHUNK, embed_pad), jnp.float32),
            [pltpu.SemaphoreType.DMA] * 2,
            [pltpu.SemaphoreType.DMA] * 2,
        ],
    )
    def gather_k(x_hbm, table_hbm, out_hbm, idx_v, rows_v, sg, sw):
        wid = lax.axis_index("s") * nc + lax.axis_index("c")
        r0 = wid * rpw
        pltpu.sync_copy(x_hbm.at[pl.ds(r0, rpw)], idx_v)

        def gather(j, p):
            return pltpu.async_copy(
                table_hbm.at[idx_v.at[pl.ds(j * CHUNK, CHUNK)]],
                rows_v.at[p],
                sg[p],
            )

        pend_g = gather(0, 0)
        pend_w = [None] * nchunk
        for j in range(nchunk):
            p = j % 2
            pend_g.wait()
            if j + 1 < nchunk:
                if j >= 1:
                    pend_w[j - 1].wait()
                pend_g = gather(j + 1, 1 - p)
            pend_w[j] = pltpu.async_copy(
                rows_v.at[p],
                out_hbm.at[pl.ds(r0 + j * CHUNK, CHUNK)],
                sw[p],
            )
        pend_w[nchunk - 2].wait()
        pend_w[nchunk - 1].wait()

    return gather_k


GB = 8


def _reformat_body(in_ref, out_ref):
    for b in range(GB):
        out_ref[b] = in_ref[pl.ds(b * SEQ, SEQ), :EMBED]


@functools.lru_cache(maxsize=None)
def _build_reformat():
    return pl.pallas_call(
        _reformat_body,
        grid=(BATCH // GB,),
        in_specs=[
            pl.BlockSpec((GB * SEQ, EMBED_PAD), lambda i: (i, 0)),
        ],
        out_specs=pl.BlockSpec((GB, SEQ, EMBED), lambda i: (i, 0, 0)),
        out_shape=jax.ShapeDtypeStruct((BATCH, SEQ, EMBED), jnp.float32),
    )


def kernel(x, table):
    gather_k = _build_gather(BATCH * SEQ, EMBED_PAD)
    table_pad = jnp.pad(table, ((0, 0), (0, EMBED_PAD - EMBED)))
    rows = gather_k(x.reshape(-1), table_pad)
    return _build_reformat()(rows)


# speedup vs baseline: 1.1606x; 1.1606x over previous
import functools

import jax
import jax.numpy as jnp
from jax import lax
from jax.experimental import pallas as pl
from jax.experimental.pallas import tpu as pltpu
from jax.experimental.pallas import tpu_sc as plsc

VOCAB = 1000
EMBED = 1000
BATCH = 1024
SEQ = 20

CHUNK = 32


@functools.lru_cache(maxsize=None)
def _build_gather(total, embed):
    info = plsc.get_sparse_core_info()
    nc, ns = info.num_cores, info.num_subcores
    nw = nc * ns
    rpw = total // nw
    assert rpw * nw == total
    nchunk = rpw // CHUNK
    assert nchunk * CHUNK == rpw

    mesh = plsc.VectorSubcoreMesh(core_axis_name="c", subcore_axis_name="s")

    @functools.partial(
        pl.kernel,
        mesh=mesh,
        out_type=jax.ShapeDtypeStruct((total, embed), jnp.float32),
        compiler_params=pltpu.CompilerParams(use_tc_tiling_on_sc=False),
        scratch_types=[
            pltpu.VMEM((rpw,), jnp.int32),
            pltpu.VMEM((2, CHUNK, embed), jnp.float32),
            [pltpu.SemaphoreType.DMA] * 2,
            [pltpu.SemaphoreType.DMA] * 2,
        ],
    )
    def gather_k(x_hbm, table_hbm, out_hbm, idx_v, rows_v, sg, sw):
        wid = lax.axis_index("s") * nc + lax.axis_index("c")
        r0 = wid * rpw
        pltpu.sync_copy(x_hbm.at[pl.ds(r0, rpw)], idx_v)

        def gather(j, p):
            return pltpu.async_copy(
                table_hbm.at[idx_v.at[pl.ds(j * CHUNK, CHUNK)]],
                rows_v.at[p],
                sg[p],
            )

        pend_g = gather(0, 0)
        pend_w = [None] * nchunk
        for j in range(nchunk):
            p = j % 2
            pend_g.wait()
            if j + 1 < nchunk:
                if j >= 1:
                    pend_w[j - 1].wait()
                pend_g = gather(j + 1, 1 - p)
            pend_w[j] = pltpu.async_copy(
                rows_v.at[p],
                out_hbm.at[pl.ds(r0 + j * CHUNK, CHUNK)],
                sw[p],
            )
        pend_w[nchunk - 2].wait()
        pend_w[nchunk - 1].wait()

    return gather_k


def kernel(x, table):
    gather_k = _build_gather(BATCH * SEQ, EMBED)
    rows = gather_k(x.reshape(-1), table)
    return rows.reshape(BATCH, SEQ, EMBED)


# speedup vs baseline: 1.1798x; 1.0165x over previous
import functools

import jax
import jax.numpy as jnp
from jax import lax
from jax.experimental import pallas as pl
from jax.experimental.pallas import tpu as pltpu
from jax.experimental.pallas import tpu_sc as plsc

VOCAB = 1000
EMBED = 1000
BATCH = 1024
SEQ = 20

UNIT = 16
RING = 5


@functools.lru_cache(maxsize=None)
def _build(batch, seq, embed):
    info = plsc.get_sparse_core_info()
    nc, ns = info.num_cores, info.num_subcores
    nw = nc * ns
    bpw = batch // nw
    assert bpw * nw == batch
    rpw = bpw * seq
    nunit = rpw // UNIT
    assert nunit * UNIT == rpw
    ring_rows = RING * UNIT
    assert ring_rows % seq == 0

    mesh = plsc.VectorSubcoreMesh(core_axis_name="c", subcore_axis_name="s")

    @functools.partial(
        pl.kernel,
        mesh=mesh,
        out_type=jax.ShapeDtypeStruct((batch, seq, embed), jnp.float32),
        compiler_params=pltpu.CompilerParams(use_tc_tiling_on_sc=False),
        scratch_types=[
            pltpu.VMEM((rpw,), jnp.int32),
            pltpu.VMEM((ring_rows, embed), jnp.float32),
            [pltpu.SemaphoreType.DMA] * RING,
            [pltpu.SemaphoreType.DMA] * 2,
        ],
    )
    def emb(x_hbm, table_hbm, out_hbm, idx_v, ring_v, sg, sw):
        wid = lax.axis_index("s") * nc + lax.axis_index("c")
        b0 = wid * bpw
        pltpu.sync_copy(x_hbm.at[pl.ds(b0 * seq, rpw)], idx_v)

        def gather(u):
            return pltpu.async_copy(
                table_hbm.at[idx_v.at[pl.ds(u * UNIT, UNIT)]],
                ring_v.at[pl.ds((u % RING) * UNIT, UNIT)],
                sg[u % RING],
            )

        pend_g = [gather(u) for u in range(RING)]
        issued = RING
        waited_g = 0
        waited_w = -1
        pend_w = [None] * bpw
        for k in range(bpw):
            last_u = (seq * k + seq - 1) // UNIT
            while waited_g <= last_u:
                pend_g[waited_g].wait()
                waited_g += 1
            pend_w[k] = pltpu.async_copy(
                ring_v.at[pl.ds((seq * k) % ring_rows, seq)],
                out_hbm.at[b0 + k],
                sw[k % 2],
            )
            while issued < nunit:
                wt = (UNIT * (issued - RING) + UNIT - 1) // seq
                if wt > k:
                    break
                while waited_w < wt:
                    waited_w += 1
                    pend_w[waited_w].wait()
                pend_g.append(gather(issued))
                issued += 1
        for k2 in range(waited_w + 1, bpw):
            pend_w[k2].wait()

    return emb


def kernel(x, table):
    emb = _build(BATCH, SEQ, EMBED)
    return emb(x.reshape(-1), table)
